# baseline (speedup 1.0000x reference)
import functools

import jax
import jax.numpy as jnp
from jax import lax
from jax.experimental import pallas as pl
from jax.experimental.pallas import tpu as pltpu
from jax.experimental.pallas import tpu_sc as plsc

B, V, E, D = 2, 50000, 150000, 128
NC, NS = 2, 16
NW = NC * NS
TPB = NW // B
EPT = 9376
C = 96
NCHT = EPT // C + 1
REM = EPT - (NCHT - 1) * C
EPT_PAD = NCHT * C
NPAIR = (NCHT - 2) // 2


@functools.cache
def _build(width, tiled):
  mesh = plsc.VectorSubcoreMesh(core_axis_name="c", subcore_axis_name="s",
                                num_cores=NC, num_subcores=NS)
  params = (pltpu.CompilerParams() if tiled else
            pltpu.CompilerParams(use_tc_tiling_on_sc=False))
  if width == D:
    out_t = jax.ShapeDtypeStruct((B * E, D), jnp.float32)
    buf_t = pltpu.VMEM((C, D), jnp.float32)
    idxn = EPT_PAD
    cw = C
  else:
    out_t = jax.ShapeDtypeStruct((B * E * 3,), jnp.float32)
    buf_t = pltpu.VMEM((C * 3,), jnp.float32)
    idxn = EPT_PAD * 3
    cw = C * 3

  @functools.partial(
      pl.kernel,
      out_type=out_t,
      mesh=mesh,
      compiler_params=params,
      scratch_types=[
          pltpu.VMEM((idxn,), jnp.int32),
          pltpu.VMEM((idxn,), jnp.int32),
          buf_t, buf_t, buf_t, buf_t, buf_t, buf_t,
          pltpu.SemaphoreType.DMA,
          pltpu.SemaphoreType.DMA,
          pltpu.SemaphoreType.DMA,
          pltpu.SemaphoreType.DMA,
      ],
  )
  def _run(i0_hbm, i1_hbm, tab_hbm, out_hbm,
           iall0, iall1, fa0, fa1, fb0, fb1, wo0, wo1, sg0, sg1, sw0, sw1):
    wid = lax.axis_index("s") * NC + lax.axis_index("c")
    b = wid // TPB
    j = wid % TPB
    idx_base = wid * idxn
    start = jnp.minimum(j * EPT, E - EPT)
    if width == D:
      mid_base = b * E + start
    else:
      mid_base = (b * E + start) * 3

    bufs = ((fa0, fb0, wo0, sg0, sw0), (fa1, fb1, wo1, sg1, sw1))

    def issue_gathers(g, p):
      fa, fb, _, sg, _ = bufs[p]
      pltpu.async_copy(tab_hbm.at[iall0.at[pl.ds(g * cw, cw)]], fa, sg)
      pltpu.async_copy(tab_hbm.at[iall1.at[pl.ds(g * cw, cw)]], fb, sg)

    def wait_gathers(p):
      fa, fb, _, sg, _ = bufs[p]
      d = iall0.at[pl.ds(0, cw)]
      pltpu.make_async_copy(tab_hbm.at[d], fa, sg).wait()
      pltpu.make_async_copy(tab_hbm.at[d], fb, sg).wait()

    def compute(p):
      fa, fb, wo, _, _ = bufs[p]
      if width == D:
        @plsc.parallel_loop(0, C, step=1, unroll=4)
        def row(r):
          for k in range(D // 16):
            wo[r, pl.ds(k * 16, 16)] = (
                fa[r, pl.ds(k * 16, 16)] + fb[r, pl.ds(k * 16, 16)]) * 0.5
      else:
        @plsc.parallel_loop(0, (C * 3) // 16, step=1, unroll=2)
        def grp(i):
          wo[pl.ds(i * 16, 16)] = (
              fa[pl.ds(i * 16, 16)] + fb[pl.ds(i * 16, 16)]) * 0.5

    def issue_writes(g, p, nrows):
      _, _, wo, _, sw = bufs[p]
      n = nrows if width == D else nrows * 3
      if width == D:
        dst = out_hbm.at[pl.ds(mid_base + g * C, n)]
      else:
        dst = out_hbm.at[pl.ds(mid_base + g * C * 3, n)]
      pltpu.async_copy(wo.at[pl.ds(0, n)], dst, sw)

    def wait_writes(p, nrows):
      _, _, wo, _, sw = bufs[p]
      n = nrows if width == D else nrows * 3
      pltpu.make_async_copy(wo.at[pl.ds(0, n)],
                            out_hbm.at[pl.ds(0, n)], sw).wait()

    pltpu.sync_copy(i0_hbm.at[pl.ds(idx_base, idxn)], iall0)
    pltpu.sync_copy(i1_hbm.at[pl.ds(idx_base, idxn)], iall1)
    issue_gathers(0, 0)
    issue_gathers(1, 1)

    def pair(i, carry):
      for p in (0, 1):
        g = 2 * i + p
        wait_gathers(p)

        @pl.when(g >= 2)
        def _():
          wait_writes(p, C)

        compute(p)
        issue_gathers(g + 2, p)
        issue_writes(g, p, C)
      return carry

    lax.fori_loop(0, NPAIR, pair, 0)

    wait_gathers(0)
    wait_writes(0, C)
    compute(0)
    issue_writes(NCHT - 2, 0, C)
    wait_gathers(1)
    wait_writes(1, C)
    compute(1)
    issue_writes(NCHT - 1, 1, REM)

    wait_writes(0, C)
    wait_writes(1, REM)

  return _run


def _tile_indices(e, offs):
  main = e[:(TPB - 1) * EPT].reshape(TPB - 1, EPT)
  last = e[E - EPT:][None, :]
  per_tile = jnp.concatenate([main, last], axis=0)
  both = per_tile[None, :, :] + offs[:, None, None]
  return jnp.pad(both, ((0, 0), (0, 0), (0, EPT_PAD - EPT)))


def _elem_indices(i):
  return (i[..., None] * 3 +
          jnp.arange(3, dtype=jnp.int32)).reshape(B, TPB, EPT_PAD * 3)


@jax.jit
def kernel(verts, feats, edges):
  offs = jnp.arange(B, dtype=jnp.int32) * V
  i0 = _tile_indices(edges[:, 0], offs)
  i1 = _tile_indices(edges[:, 1], offs)
  mid_f = _build(D, True)(i0.reshape(-1), i1.reshape(-1), feats)
  mid_v = _build(3, False)(_elem_indices(i0).reshape(-1),
                           _elem_indices(i1).reshape(-1),
                           verts.reshape(-1))
  new_feats = jnp.concatenate(
      [feats.reshape(B, V, D), mid_f.reshape(B, E, D)], axis=1).reshape(-1, D)
  new_verts = jnp.concatenate([verts, mid_v.reshape(B, E, 3)], axis=1)
  return new_verts, new_feats


# speedup vs baseline: 1.7875x; 1.7875x over previous
import functools

import jax
import jax.numpy as jnp
from jax import lax
from jax.experimental import pallas as pl
from jax.experimental.pallas import tpu as pltpu
from jax.experimental.pallas import tpu_sc as plsc

B, V, E, D = 2, 50000, 150000, 128
NC, NS = 2, 16
NW = NC * NS
TPB = NW // B
EPT = 9376
C = 96
NCHT = EPT // C + 1
REM = EPT - (NCHT - 1) * C
EPT_PAD = NCHT * C
NPAIR = (NCHT - 2) // 2
RPT = 3128
NCC = RPT // C
REMC = RPT - NCC * C
NCP = NCC // 2
OUTR = B * (V + E)

_mesh_kw = dict(core_axis_name="c", subcore_axis_name="s",
                num_cores=NC, num_subcores=NS)


@functools.cache
def _build_feats():
  mesh = plsc.VectorSubcoreMesh(**_mesh_kw)

  @functools.partial(
      pl.kernel,
      out_type=jax.ShapeDtypeStruct((OUTR, D), jnp.float32),
      mesh=mesh,
      scratch_types=[
          pltpu.VMEM((EPT_PAD,), jnp.int32),
          pltpu.VMEM((EPT_PAD,), jnp.int32),
          pltpu.VMEM((C, D), jnp.float32),
          pltpu.VMEM((C, D), jnp.float32),
          pltpu.VMEM((C, D), jnp.float32),
          pltpu.VMEM((C, D), jnp.float32),
          pltpu.VMEM((C, D), jnp.float32),
          pltpu.VMEM((C, D), jnp.float32),
          pltpu.SemaphoreType.DMA,
          pltpu.SemaphoreType.DMA,
          pltpu.SemaphoreType.DMA,
          pltpu.SemaphoreType.DMA,
          pltpu.SemaphoreType.DMA,
          pltpu.SemaphoreType.DMA,
          pltpu.SemaphoreType.DMA,
          pltpu.SemaphoreType.DMA,
      ],
  )
  def _run(i0_hbm, i1_hbm, tab_hbm, out_hbm,
           iall0, iall1, fa0, fa1, fb0, fb1, wo0, wo1,
           sg0, sg1, sw0, sw1, sci0, sci1, sco0, sco1):
    wid = lax.axis_index("s") * NC + lax.axis_index("c")
    b = wid // TPB
    j = wid % TPB
    idx_base = wid * EPT_PAD
    start = jnp.minimum(j * EPT, E - EPT)
    mid_base = b * (V + E) + V + start
    cstart = jnp.minimum(j * RPT, V - RPT)
    cin_base = b * V + cstart
    cout_base = b * (V + E) + cstart

    bufs = ((fa0, fb0, wo0, sg0, sw0), (fa1, fb1, wo1, sg1, sw1))

    def issue_gathers(g, p):
      fa, fb, _, sg, _ = bufs[p]
      pltpu.async_copy(tab_hbm.at[iall0.at[pl.ds(g * C, C)]], fa, sg)
      pltpu.async_copy(tab_hbm.at[iall1.at[pl.ds(g * C, C)]], fb, sg)

    def wait_gathers(p):
      fa, fb, _, sg, _ = bufs[p]
      d = iall0.at[pl.ds(0, C)]
      pltpu.make_async_copy(tab_hbm.at[d], fa, sg).wait()
      pltpu.make_async_copy(tab_hbm.at[d], fb, sg).wait()

    def compute(p):
      fa, fb, wo, _, _ = bufs[p]

      @plsc.parallel_loop(0, C, step=1, unroll=4)
      def row(r):
        for k in range(D // 16):
          wo[r, pl.ds(k * 16, 16)] = (
              fa[r, pl.ds(k * 16, 16)] + fb[r, pl.ds(k * 16, 16)]) * 0.5

    def issue_writes(g, p, nrows):
      _, _, wo, _, sw = bufs[p]
      pltpu.async_copy(wo.at[pl.ds(0, nrows)],
                       out_hbm.at[pl.ds(mid_base + g * C, nrows)], sw)

    def wait_writes(p, nrows):
      _, _, wo, _, sw = bufs[p]
      pltpu.make_async_copy(wo.at[pl.ds(0, nrows)],
                            out_hbm.at[pl.ds(0, nrows)], sw).wait()

    pltpu.sync_copy(i0_hbm.at[pl.ds(idx_base, EPT_PAD)], iall0)
    pltpu.sync_copy(i1_hbm.at[pl.ds(idx_base, EPT_PAD)], iall1)
    issue_gathers(0, 0)
    issue_gathers(1, 1)

    def pair(i, carry):
      for p in (0, 1):
        g = 2 * i + p
        wait_gathers(p)

        @pl.when(g >= 2)
        def _():
          wait_writes(p, C)

        compute(p)
        issue_gathers(g + 2, p)
        issue_writes(g, p, C)
      return carry

    lax.fori_loop(0, NPAIR, pair, 0)

    wait_gathers(0)
    wait_writes(0, C)
    compute(0)
    issue_writes(NCHT - 2, 0, C)
    wait_gathers(1)
    wait_writes(1, C)
    compute(1)
    issue_writes(NCHT - 1, 1, REM)

    cbufs = ((fa0, sci0, sco0), (fa1, sci1, sco1))

    def cpair(jj, carry):
      for p in (0, 1):
        cf, sci, sco = cbufs[p]
        c = 2 * jj + p

        @pl.when(c >= 2)
        def _():
          pltpu.make_async_copy(cf, out_hbm.at[pl.ds(0, C)], sco).wait()

        pltpu.async_copy(tab_hbm.at[pl.ds(cin_base + c * C, C)], cf, sci)
      for p in (0, 1):
        cf, sci, sco = cbufs[p]
        c = 2 * jj + p
        pltpu.make_async_copy(tab_hbm.at[pl.ds(0, C)], cf, sci).wait()
        pltpu.async_copy(cf, out_hbm.at[pl.ds(cout_base + c * C, C)], sco)
      return carry

    lax.fori_loop(0, NCP, cpair, 0)

    for p in (0, 1):
      cf, _, sco = cbufs[p]
      pltpu.make_async_copy(cf, out_hbm.at[pl.ds(0, C)], sco).wait()

    pltpu.sync_copy(tab_hbm.at[pl.ds(cin_base + NCC * C, REMC)],
                    fa0.at[pl.ds(0, REMC)])
    pltpu.sync_copy(fa0.at[pl.ds(0, REMC)],
                    out_hbm.at[pl.ds(cout_base + NCC * C, REMC)])

    wait_writes(0, C)
    wait_writes(1, REM)

  return _run


CV = C * 3
NG = CV // 16


@functools.cache
def _build_verts():
  mesh = plsc.VectorSubcoreMesh(**_mesh_kw)

  @functools.partial(
      pl.kernel,
      out_type=jax.ShapeDtypeStruct((B * E * 3,), jnp.float32),
      mesh=mesh,
      compiler_params=pltpu.CompilerParams(use_tc_tiling_on_sc=False,
                                           needs_layout_passes=False),
      scratch_types=[
          pltpu.VMEM((EPT_PAD,), jnp.int32),
          pltpu.VMEM((EPT_PAD,), jnp.int32),
          pltpu.VMEM((CV,), jnp.int32),
          pltpu.VMEM((CV,), jnp.int32),
          pltpu.VMEM((CV,), jnp.int32),
          pltpu.VMEM((CV,), jnp.int32),
          pltpu.VMEM((CV,), jnp.float32),
          pltpu.VMEM((CV,), jnp.float32),
          pltpu.VMEM((CV,), jnp.float32),
          pltpu.VMEM((CV,), jnp.float32),
          pltpu.VMEM((CV,), jnp.float32),
          pltpu.VMEM((CV,), jnp.float32),
          pltpu.SemaphoreType.DMA,
          pltpu.SemaphoreType.DMA,
          pltpu.SemaphoreType.DMA,
          pltpu.SemaphoreType.DMA,
      ],
  )
  def _run(i0_hbm, i1_hbm, tab_hbm, out_hbm,
           iall0, iall1, ie0a, ie0b, ie1a, ie1b,
           fa0, fa1, fb0, fb1, wo0, wo1, sg0, sg1, sw0, sw1):
    wid = lax.axis_index("s") * NC + lax.axis_index("c")
    b = wid // TPB
    j = wid % TPB
    idx_base = wid * EPT_PAD
    start = jnp.minimum(j * EPT, E - EPT)
    mid_base = (b * E + start) * 3

    io = lax.iota(jnp.int32, 16)
    rowp = []
    colp = []
    for t in range(3):
      q = io + 16 * t
      r = lax.shift_right_logical(q * 21846, 16)
      rowp.append(r)
      colp.append(q - r * 3)

    bufs = ((fa0, fb0, ie0a, ie0b, wo0, sg0, sw0),
            (fa1, fb1, ie1a, ie1b, wo1, sg1, sw1))

    def issue_gathers(g, p):
      fa, fb, iea, ieb, _, sg, _ = bufs[p]
      for i in range(NG):
        m, t = divmod(i, 3)
        rows = g * C + (rowp[t] + 16 * m)
        r0 = plsc.load_gather(iall0, [rows])
        r1 = plsc.load_gather(iall1, [rows])
        iea[pl.ds(i * 16, 16)] = r0 * 3 + colp[t]
        ieb[pl.ds(i * 16, 16)] = r1 * 3 + colp[t]
      pltpu.async_copy(tab_hbm.at[iea], fa, sg)
      pltpu.async_copy(tab_hbm.at[ieb], fb, sg)

    def wait_gathers(p):
      fa, fb, iea, ieb, _, sg, _ = bufs[p]
      pltpu.make_async_copy(tab_hbm.at[iea], fa, sg).wait()
      pltpu.make_async_copy(tab_hbm.at[ieb], fb, sg).wait()

    def compute(p):
      fa, fb, _, _, wo, _, _ = bufs[p]

      @plsc.parallel_loop(0, NG, step=1, unroll=2)
      def grp(i):
        wo[pl.ds(i * 16, 16)] = (
            fa[pl.ds(i * 16, 16)] + fb[pl.ds(i * 16, 16)]) * 0.5

    def issue_writes(g, p, nrows):
      _, _, _, _, wo, _, sw = bufs[p]
      pltpu.async_copy(wo.at[pl.ds(0, nrows * 3)],
                       out_hbm.at[pl.ds(mid_base + g * CV, nrows * 3)], sw)

    def wait_writes(p, nrows):
      _, _, _, _, wo, _, sw = bufs[p]
      pltpu.make_async_copy(wo.at[pl.ds(0, nrows * 3)],
                            out_hbm.at[pl.ds(0, nrows * 3)], sw).wait()

    pltpu.sync_copy(i0_hbm.at[pl.ds(idx_base, EPT_PAD)], iall0)
    pltpu.sync_copy(i1_hbm.at[pl.ds(idx_base, EPT_PAD)], iall1)
    issue_gathers(0, 0)
    issue_gathers(1, 1)

    def pair(i, carry):
      for p in (0, 1):
        g = 2 * i + p
        wait_gathers(p)

        @pl.when(g >= 2)
        def _():
          wait_writes(p, C)

        compute(p)
        issue_gathers(g + 2, p)
        issue_writes(g, p, C)
      return carry

    lax.fori_loop(0, NPAIR, pair, 0)

    wait_gathers(0)
    wait_writes(0, C)
    compute(0)
    issue_writes(NCHT - 2, 0, C)
    wait_gathers(1)
    wait_writes(1, C)
    compute(1)
    issue_writes(NCHT - 1, 1, REM)

    wait_writes(0, C)
    wait_writes(1, REM)

  return _run


def _tile_indices(e, offs):
  main = e[:(TPB - 1) * EPT].reshape(TPB - 1, EPT)
  last = e[E - EPT:][None, :]
  per_tile = jnp.concatenate([main, last], axis=0)
  both = per_tile[None, :, :] + offs[:, None, None]
  return jnp.pad(both, ((0, 0), (0, 0), (0, EPT_PAD - EPT))).reshape(-1)


@jax.jit
def kernel(verts, feats, edges):
  offs = jnp.arange(B, dtype=jnp.int32) * V
  i0 = _tile_indices(edges[:, 0], offs)
  i1 = _tile_indices(edges[:, 1], offs)
  new_feats = _build_feats()(i0, i1, feats)
  mid_v = _build_verts()(i0, i1, verts.reshape(-1))
  new_verts = jnp.concatenate([verts, mid_v.reshape(B, E, 3)], axis=1)
  return new_verts, new_feats
